# baseline (speedup 1.0000x reference)
import functools

import jax
import jax.numpy as jnp
from jax import lax
from jax.experimental import pallas as pl
from jax.experimental.pallas import tpu as pltpu
from jax.experimental.pallas import tpu_sc as plsc

VOCAB = 49408
EMBED = 768
MAX_POS = 77
BATCH = 1024
SEQ = 77

NC, NS, L = 2, 16, 16
NW = NC * NS
BPW = BATCH // NW
DV = EMBED // L


def _body(ids_hbm, tok_hbm, pos_hbm, out_hbm, idx_v, pos_v, buf, gsem):
    wid = lax.axis_index("s") * NC + lax.axis_index("c")
    b0 = wid * BPW
    pltpu.sync_copy(ids_hbm.at[pl.ds(wid * SEQ * BPW, SEQ * BPW)], idx_v)
    pltpu.sync_copy(pos_hbm, pos_v)

    def chunk(s, _):
        pltpu.async_copy(tok_hbm.at[idx_v.at[pl.ds(s * BPW, BPW)]], buf, gsem).wait()
        def dloop(d, _):
            sl = pl.ds(d * L, L)
            pvec = pos_v[s, sl]
            for r in range(BPW):
                buf[r, sl] = buf[r, sl] + pvec
            return 0
        lax.fori_loop(0, DV, dloop, 0, unroll=False)
        pltpu.sync_copy(buf, out_hbm.at[pl.ds(b0, BPW), pl.ds(s * EMBED, EMBED)])
        return 0

    lax.fori_loop(0, SEQ, chunk, 0, unroll=False)


@jax.jit
def _run(ids_t, token_embedding, position_embedding):
    mesh = plsc.VectorSubcoreMesh(
        core_axis_name="c", subcore_axis_name="s", num_cores=NC, num_subcores=NS)
    f = pl.kernel(
        _body,
        out_type=jax.ShapeDtypeStruct((BATCH, SEQ * EMBED), jnp.float32),
        mesh=mesh,
        scratch_types=[
            pltpu.VMEM((SEQ * BPW,), jnp.int32),
            pltpu.VMEM((MAX_POS, EMBED), jnp.float32),
            pltpu.VMEM((BPW, EMBED), jnp.float32),
            pltpu.SemaphoreType.DMA,
        ],
    )
    out2d = f(ids_t, token_embedding, position_embedding)
    return out2d.reshape(BATCH, SEQ, EMBED)


def kernel(input_ids, token_embedding, position_embedding):
    ids_p = (input_ids.astype(jnp.int32)
             .reshape(NW, BPW, SEQ).transpose(0, 2, 1).reshape(-1))
    return _run(ids_p, token_embedding, position_embedding)


# speedup vs baseline: 1.2649x; 1.2649x over previous
import jax
import jax.numpy as jnp
from jax import lax
from jax.experimental import pallas as pl
from jax.experimental.pallas import tpu as pltpu
from jax.experimental.pallas import tpu_sc as plsc

VOCAB = 49408
EMBED = 768
MAX_POS = 77
BATCH = 1024
SEQ = 77

NC, NS, L = 2, 16, 16
NW = NC * NS
BPW = BATCH // NW
DV = EMBED // L
NB = 4


def _body(ids_hbm, tok_hbm, pos_hbm, out_hbm,
          idx_v, pos_st, buf0, buf1, buf2, buf3,
          g0, g1, g2, g3, s0, s1, s2, s3):
    bufs = (buf0, buf1, buf2, buf3)
    gsems = (g0, g1, g2, g3)
    ssems = (s0, s1, s2, s3)
    wid = lax.axis_index("s") * NC + lax.axis_index("c")
    b0 = wid * BPW
    pltpu.sync_copy(ids_hbm.at[pl.ds(wid * SEQ * BPW, SEQ * BPW)], idx_v)

    def fire_g(s, b):
        pltpu.async_copy(
            tok_hbm.at[idx_v.at[pl.ds(s * BPW, BPW)]], bufs[b], gsems[b])
        pltpu.async_copy(
            pos_hbm.at[pl.ds(s * EMBED, EMBED)], pos_st.at[b], gsems[b])

    def wait_g(b):
        pltpu.make_async_copy(
            tok_hbm.at[pl.ds(0, BPW)], bufs[b], gsems[b]).wait()
        pltpu.make_async_copy(
            pos_hbm.at[pl.ds(0, EMBED)], pos_st.at[b], gsems[b]).wait()

    def fire_s(s, b):
        pltpu.async_copy(
            bufs[b], out_hbm.at[pl.ds(b0, BPW), pl.ds(s * EMBED, EMBED)],
            ssems[b])

    def wait_s(b):
        pltpu.make_async_copy(
            bufs[b], out_hbm.at[pl.ds(b0, BPW), pl.ds(0, EMBED)],
            ssems[b]).wait()

    def add_pos(s, b):
        buf = bufs[b]
        def dloop(d, _):
            sl = pl.ds(d * L, L)
            pvec = pos_st[b, sl]
            for r in range(BPW):
                buf[r, sl] = buf[r, sl] + pvec
            return 0
        lax.fori_loop(0, DV, dloop, 0, unroll=False)

    def process(s, b, wait_scatter):
        wait_g(b)
        add_pos(s, b)
        fire_s(s, b)
        nxt = s + NB - 2
        if wait_scatter:
            wait_s((b + NB - 2) % NB)
        fire_g(nxt, (b + NB - 2) % NB)

    fire_g(0, 0)
    fire_g(1, 1)
    process(0, 0, wait_scatter=False)
    process(1, 1, wait_scatter=False)
    process(2, 2, wait_scatter=True)
    process(3, 3, wait_scatter=True)

    def group(g, _):
        sb = g * NB
        for b in range(NB):
            s = sb + b
            wait_g(b)
            add_pos(s, b)
            fire_s(s, b)
            nxt = s + NB - 2
            bn = (b + NB - 2) % NB

            @pl.when(nxt < SEQ)
            def _():
                wait_s(bn)
                fire_g(nxt, bn)
            del _
        return 0

    lax.fori_loop(1, (SEQ - 1) // NB, group, 0, unroll=False)

    wait_g(0)
    add_pos(SEQ - 1, 0)
    fire_s(SEQ - 1, 0)
    for b in range(NB):
        wait_s(b)


@jax.jit
def _run(ids_p, token_embedding, pos_flat):
    mesh = plsc.VectorSubcoreMesh(
        core_axis_name="c", subcore_axis_name="s", num_cores=NC, num_subcores=NS)
    f = pl.kernel(
        _body,
        out_type=jax.ShapeDtypeStruct((BATCH, SEQ * EMBED), jnp.float32),
        mesh=mesh,
        scratch_types=[
            pltpu.VMEM((SEQ * BPW,), jnp.int32),
            pltpu.VMEM((NB, EMBED), jnp.float32),
        ] + [pltpu.VMEM((BPW, EMBED), jnp.float32)] * NB
          + [pltpu.SemaphoreType.DMA] * (2 * NB),
    )
    out2d = f(ids_p, token_embedding, pos_flat)
    return out2d.reshape(BATCH, SEQ, EMBED)


def kernel(input_ids, token_embedding, position_embedding):
    ids_p = (input_ids.astype(jnp.int32)
             .reshape(NW, BPW, SEQ).transpose(0, 2, 1).reshape(-1))
    return _run(ids_p, token_embedding, position_embedding.reshape(-1))
